# baseline (speedup 1.0000x reference)
import functools

import jax
import jax.numpy as jnp
from jax import lax
from jax.experimental import pallas as pl
from jax.experimental.pallas import tpu as pltpu
from jax.experimental.pallas import tpu_sc as plsc

NC, NS = 2, 16
NW = NC * NS
CHUNK = 128


def _sc_gather_sum(poi_table, cat_table, poi_idx2, cat_idx2):
    n_rows, IW = poi_idx2.shape
    N = n_rows * IW
    D = poi_table.shape[1]
    n_per_w = N // NW
    n_chunks = n_per_w // CHUNK
    K = 8
    n_groups = n_chunks // K
    mesh = plsc.VectorSubcoreMesh(core_axis_name="c", subcore_axis_name="s")

    @functools.partial(
        pl.kernel,
        out_type=jax.ShapeDtypeStruct((N // 2, 2 * D), jnp.float32),
        mesh=mesh,
        compiler_params=pltpu.CompilerParams(use_tc_tiling_on_sc=False),
        scratch_types=[
            pltpu.VMEM((n_chunks, CHUNK), jnp.int32),
            pltpu.VMEM((n_chunks, CHUNK), jnp.int32),
            pltpu.VMEM((K, CHUNK, D), jnp.float32),
            pltpu.SemaphoreType.DMA,
            pltpu.SemaphoreType.DMA,
            pltpu.SemaphoreType.DMA,
        ],
    )
    def k(poi_t, cat_t, pidx_h, cidx_h, out_h, pidx_v, cidx_v, bufs,
          sem_c, sem_p, sem_o):
        wid = lax.axis_index("s") * NC + lax.axis_index("c")
        crow = wid * n_chunks
        rowbase = (wid % 16) * n_per_w
        col0 = (wid // 16) * D
        pltpu.sync_copy(pidx_h.at[pl.ds(crow, n_chunks)], pidx_v)
        pltpu.sync_copy(cidx_h.at[pl.ds(crow, n_chunks)], cidx_v)

        def out_descs(g):
            return [
                pltpu.make_async_copy(
                    bufs.at[s],
                    out_h.at[pl.ds(rowbase + (g * K + s) * CHUNK, CHUNK),
                             pl.ds(col0, D)],
                    sem_o)
                for s in range(K)
            ]

        def body(g, carry):
            @pl.when(g > 0)
            def _():
                for d in out_descs(g - 1):
                    d.wait()

            cats = [
                pltpu.async_copy(
                    cat_t.at[cidx_v.at[g * K + s]], bufs.at[s], sem_c)
                for s in range(K)
            ]
            for d in cats:
                d.wait()
            pois = [
                pltpu.async_copy(
                    poi_t.at[pidx_v.at[g * K + s]], bufs.at[s], sem_p,
                    add=True)
                for s in range(K)
            ]
            for d in pois:
                d.wait()
            for d in out_descs(g):
                d.start()
            return carry

        lax.fori_loop(0, n_groups, body, 0)
        for d in out_descs(n_groups - 1):
            d.wait()

    return k(poi_table, cat_table, poi_idx2, cat_idx2)


def _tc_dense(packed, hour3, W, hour_pad, b, B_, L_):
    M, D2 = packed.shape
    D = D2 // 2
    N = 2 * M
    TOK = 800
    RB = TOK // L_
    n_steps = N // TOK
    half_steps = n_steps // 2
    H = hour_pad.shape[0]
    TBp = TOK

    def body(x_ref, h_ref, w_ref, hp_ref, b_ref, o_ref):
        half = pl.program_id(0) // half_steps
        x2 = x_ref[...]
        x = jnp.where(half == 0, x2[:, :D], x2[:, D:])
        h = h_ref[0, 0, :]
        Wfull = w_ref[...]
        hp = lax.dot_general(
            hp_ref[...], Wfull[:, D:], (((1,), (1,)), ((), ())),
            preferred_element_type=jnp.float32)
        hp = hp + b_ref[...][None, :]
        oh = (h[:, None] == lax.broadcasted_iota(jnp.int32, (TOK, H), 1)
              ).astype(jnp.float32)
        y = lax.dot_general(
            x, Wfull[:, :D], (((1,), (1,)), ((), ())),
            preferred_element_type=jnp.float32)
        y = y + lax.dot_general(
            oh, hp, (((1,), (0,)), ((), ())),
            preferred_element_type=jnp.float32)
        o_ref[...] = jnp.tanh(y).reshape(RB, L_, D)

    return pl.pallas_call(
        body,
        grid=(n_steps,),
        in_specs=[
            pl.BlockSpec((TBp, D2), lambda j: (lax.rem(j, half_steps), 0)),
            pl.BlockSpec((1, 1, TOK), lambda j: (j, 0, 0)),
            pl.BlockSpec(W.shape, lambda j: (0, 0)),
            pl.BlockSpec(hour_pad.shape, lambda j: (0, 0)),
            pl.BlockSpec(b.shape, lambda j: (0,)),
        ],
        out_specs=pl.BlockSpec((RB, L_, D), lambda j: (j, 0, 0)),
        out_shape=jax.ShapeDtypeStruct((B_, L_, D), jnp.float32),
    )(packed, hour3, W, hour_pad, b)


def kernel(poi_seq, category_seq, hour_seq, poi_table, cat_table, hour_table, W, b):
    B_, L_ = poi_seq.shape
    D = poi_table.shape[1]
    N = B_ * L_

    pidx2 = poi_seq.astype(jnp.int32).reshape(N // CHUNK, CHUNK)
    cidx2 = category_seq.astype(jnp.int32).reshape(N // CHUNK, CHUNK)
    packed = _sc_gather_sum(poi_table, cat_table, pidx2, cidx2)

    hour3 = hour_seq.astype(jnp.int32).reshape(N // 800, 1, 800)
    hour_pad = jnp.pad(hour_table, ((0, 32 - hour_table.shape[0]), (0, 0)))
    return _tc_dense(packed, hour3, W, hour_pad, b, B_, L_)


# speedup vs baseline: 1.6555x; 1.6555x over previous
import functools

import jax
import jax.numpy as jnp
from jax import lax
from jax.experimental import pallas as pl
from jax.experimental.pallas import tpu as pltpu
from jax.experimental.pallas import tpu_sc as plsc

NC, NS = 2, 16
NW = NC * NS
CHUNK = 128


def _sc_gather_sum(poi_flat, cat_flat, D, poi_idx2, cat_idx2, K):
    n_rows, IW = poi_idx2.shape
    N = n_rows * IW
    n_per_w = N // NW
    n_chunks = n_per_w // CHUNK
    n_groups = n_chunks // K
    mesh = plsc.VectorSubcoreMesh(core_axis_name="c", subcore_axis_name="s")

    @functools.partial(
        pl.kernel,
        out_type=jax.ShapeDtypeStruct((N // 2, 2 * D), jnp.float32),
        mesh=mesh,
        compiler_params=pltpu.CompilerParams(use_tc_tiling_on_sc=False),
        scratch_types=[
            pltpu.VMEM((n_chunks, CHUNK), jnp.int32),
            pltpu.VMEM((n_chunks, CHUNK), jnp.int32),
            pltpu.VMEM((K, CHUNK, D), jnp.float32),
            pltpu.SemaphoreType.DMA,
            pltpu.SemaphoreType.DMA,
            pltpu.SemaphoreType.DMA,
        ],
    )
    def k(poi_f, cat_f, pidx_h, cidx_h, out_h, pidx_v, cidx_v, bufs,
          sem_c, sem_p, sem_o):
        poi_t = poi_f
        cat_t = cat_f
        wid = lax.axis_index("s") * NC + lax.axis_index("c")
        crow = wid * n_chunks
        rowbase = (wid % 16) * n_per_w
        col0 = (wid // 16) * D
        pltpu.sync_copy(pidx_h.at[pl.ds(crow, n_chunks)], pidx_v)
        pltpu.sync_copy(cidx_h.at[pl.ds(crow, n_chunks)], cidx_v)

        def out_descs(g):
            return [
                pltpu.make_async_copy(
                    bufs.at[s],
                    out_h.at[pl.ds(rowbase + (g * K + s) * CHUNK, CHUNK),
                             pl.ds(col0, D)],
                    sem_o)
                for s in range(K)
            ]

        def body(g, carry):
            @pl.when(g > 0)
            def _():
                for d in out_descs(g - 1):
                    d.wait()

            cats = [
                pltpu.async_copy(
                    cat_t.at[cidx_v.at[g * K + s]], bufs.at[s], sem_c)
                for s in range(K)
            ]
            for d in cats:
                d.wait()
            pois = [
                pltpu.async_copy(
                    poi_t.at[pidx_v.at[g * K + s]], bufs.at[s], sem_p,
                    add=True)
                for s in range(K)
            ]
            for d in pois:
                d.wait()
            for d in out_descs(g):
                d.start()
            return carry

        lax.fori_loop(0, n_groups, body, 0)
        for d in out_descs(n_groups - 1):
            d.wait()

    return k(poi_flat, cat_flat, poi_idx2, cat_idx2)


def _tc_dense(packed, hourT3, W, hour_pad, b, B_, L_, slab, prev):
    M, D2 = packed.shape
    D = D2 // 2
    slabL = L_ // 2
    qL = slabL // 2
    H = hour_pad.shape[0]

    def body(x_ref, h_ref, w_ref, hp_ref, b_ref, *refs):
        o_ref = refs[-1]
        half = pl.program_id(0) // qL
        x2 = x_ref[...]
        x = jnp.where(half == 0, x2[:, :D], x2[:, D:])
        Wfull = w_ref[...]
        hp = lax.dot_general(
            hp_ref[...], Wfull[:, D:], (((1,), (1,)), ((), ())),
            preferred_element_type=jnp.float32)
        hp = hp + b_ref[...][None, :]
        h = h_ref[0, 0, :]
        oh = (h[:, None] == lax.broadcasted_iota(jnp.int32, (B_, H), 1)
              ).astype(jnp.float32)
        yT = lax.dot_general(
            Wfull[:, :D], x, (((1,), (1,)), ((), ())),
            preferred_element_type=jnp.float32)
        yT = yT + lax.dot_general(
            hp, oh, (((0,), (1,)), ((), ())),
            preferred_element_type=jnp.float32)
        o_ref[0, 0, :, :] = jnp.tanh(yT)

    in_specs = [
        pl.BlockSpec((B_, D2), lambda j: (lax.rem(j, qL), 0)),
        pl.BlockSpec((1, 1, B_), lambda j: (slab * slabL + j, 0, 0)),
        pl.BlockSpec(W.shape, lambda j: (0, 0)),
        pl.BlockSpec(hour_pad.shape, lambda j: (0, 0)),
        pl.BlockSpec(b.shape, lambda j: (0,)),
    ]
    inputs = [packed, hourT3, W, hour_pad, b]
    aliases = {}
    if prev is not None:
        in_specs.append(pl.BlockSpec(memory_space=pltpu.MemorySpace.HBM))
        inputs.append(prev)
        aliases = {5: 0}
    return pl.pallas_call(
        body,
        grid=(slabL,),
        in_specs=in_specs,
        out_specs=pl.BlockSpec((1, 1, D, B_), lambda j: (slab, j, 0, 0)),
        out_shape=jax.ShapeDtypeStruct((2, slabL, D, B_), jnp.float32),
        input_output_aliases=aliases,
    )(*inputs)


def kernel(poi_seq, category_seq, hour_seq, poi_table, cat_table, hour_table, W, b):
    B_, L_ = poi_seq.shape
    D = poi_table.shape[1]
    N = B_ * L_
    irows = N // 2 // CHUNK

    pidx2 = poi_seq.astype(jnp.int32).T.reshape(N // CHUNK, CHUNK)
    cidx2 = category_seq.astype(jnp.int32).T.reshape(N // CHUNK, CHUNK)
    hourT3 = hour_seq.astype(jnp.int32).T.reshape(L_, 1, B_)
    hour_pad = jnp.pad(hour_table, ((0, 32 - hour_table.shape[0]), (0, 0)))

    packs = [
        _sc_gather_sum(poi_table, cat_table, D,
                       pidx2[s * irows:(s + 1) * irows],
                       cidx2[s * irows:(s + 1) * irows], K=10)
        for s in range(2)
    ]
    outT = _tc_dense(packs[0], hourT3, W, hour_pad, b, B_, L_, 0, None)
    outT = _tc_dense(packs[1], hourT3, W, hour_pad, b, B_, L_, 1, outT)
    return outT.reshape(L_, D, B_).transpose(2, 0, 1)


# speedup vs baseline: 1.7517x; 1.0581x over previous
import functools

import jax
import jax.numpy as jnp
from jax import lax
from jax.experimental import pallas as pl
from jax.experimental.pallas import tpu as pltpu
from jax.experimental.pallas import tpu_sc as plsc

NC, NS = 2, 16
NW = NC * NS
CHUNK = 128


def _sc_gather_sum(poi_table, cat_table, D, poi_idx2, cat_idx2, K):
    n_rows, IW = poi_idx2.shape
    N = n_rows * IW
    n_per_w = N // NW
    n_chunks = n_per_w // CHUNK
    n_groups = n_chunks // K
    mesh = plsc.VectorSubcoreMesh(core_axis_name="c", subcore_axis_name="s")

    @functools.partial(
        pl.kernel,
        out_type=jax.ShapeDtypeStruct((N // 2, 2 * D), jnp.float32),
        mesh=mesh,
        compiler_params=pltpu.CompilerParams(use_tc_tiling_on_sc=False),
        scratch_types=[
            pltpu.VMEM((n_chunks, CHUNK), jnp.int32),
            pltpu.VMEM((n_chunks, CHUNK), jnp.int32),
            pltpu.VMEM((K, CHUNK, D), jnp.float32),
            pltpu.SemaphoreType.DMA,
            pltpu.SemaphoreType.DMA,
            pltpu.SemaphoreType.DMA,
        ],
    )
    def k(poi_t, cat_t, pidx_h, cidx_h, out_h, pidx_v, cidx_v, bufs,
          sem_c, sem_p, sem_o):
        wid = lax.axis_index("s") * NC + lax.axis_index("c")
        crow = wid * n_chunks
        rowbase = (wid % 16) * n_per_w
        col0 = (wid // 16) * D
        pltpu.sync_copy(pidx_h.at[pl.ds(crow, n_chunks)], pidx_v)
        pltpu.sync_copy(cidx_h.at[pl.ds(crow, n_chunks)], cidx_v)

        def out_descs(g):
            return [
                pltpu.make_async_copy(
                    bufs.at[s],
                    out_h.at[pl.ds(rowbase + (g * K + s) * CHUNK, CHUNK),
                             pl.ds(col0, D)],
                    sem_o)
                for s in range(K)
            ]

        def body(g, carry):
            @pl.when(g > 0)
            def _():
                for d in out_descs(g - 1):
                    d.wait()

            cats = [
                pltpu.async_copy(
                    cat_t.at[cidx_v.at[g * K + s]], bufs.at[s], sem_c)
                for s in range(K)
            ]
            for d in cats:
                d.wait()
            pois = [
                pltpu.async_copy(
                    poi_t.at[pidx_v.at[g * K + s]], bufs.at[s], sem_p,
                    add=True)
                for s in range(K)
            ]
            for d in pois:
                d.wait()
            for d in out_descs(g):
                d.start()
            return carry

        lax.fori_loop(0, n_groups, body, 0)
        for d in out_descs(n_groups - 1):
            d.wait()

    return k(poi_table, cat_table, poi_idx2, cat_idx2)


def _tc_dense(packed, hourT3, W, hour_pad, b, B_, L_):
    M, D2 = packed.shape
    D = D2 // 2
    halfL = L_ // 2
    H = hour_pad.shape[0]

    def body(x_ref, hl_ref, hr_ref, w_ref, hp_ref, b_ref, o_ref):
        x2 = x_ref[...]
        Wfull = w_ref[...]
        hp = lax.dot_general(
            hp_ref[...], Wfull[:, D:], (((1,), (1,)), ((), ())),
            preferred_element_type=jnp.float32)
        hp = hp + b_ref[...][None, :]
        iota = lax.broadcasted_iota(jnp.int32, (B_, H), 1)
        for half, h_ref in ((0, hl_ref), (1, hr_ref)):
            h = h_ref[0, 0, :]
            oh = (h[:, None] == iota).astype(jnp.float32)
            yT = lax.dot_general(
                Wfull[:, :D], x2[:, half * D:(half + 1) * D],
                (((1,), (1,)), ((), ())),
                preferred_element_type=jnp.float32)
            yT = yT + lax.dot_general(
                hp, oh, (((0,), (1,)), ((), ())),
                preferred_element_type=jnp.float32)
            o_ref[half, 0, :, :] = jnp.tanh(yT)

    return pl.pallas_call(
        body,
        grid=(halfL,),
        in_specs=[
            pl.BlockSpec((B_, D2), lambda j: (j, 0)),
            pl.BlockSpec((1, 1, B_), lambda j: (j, 0, 0)),
            pl.BlockSpec((1, 1, B_), lambda j: (j + halfL, 0, 0)),
            pl.BlockSpec(W.shape, lambda j: (0, 0)),
            pl.BlockSpec(hour_pad.shape, lambda j: (0, 0)),
            pl.BlockSpec(b.shape, lambda j: (0,)),
        ],
        out_specs=pl.BlockSpec((2, 1, D, B_), lambda j: (0, j, 0, 0)),
        out_shape=jax.ShapeDtypeStruct((2, halfL, D, B_), jnp.float32),
    )(packed, hourT3, hourT3, W, hour_pad, b)


def kernel(poi_seq, category_seq, hour_seq, poi_table, cat_table, hour_table, W, b):
    B_, L_ = poi_seq.shape
    D = poi_table.shape[1]
    N = B_ * L_

    pidx2 = poi_seq.astype(jnp.int32).T.reshape(N // CHUNK, CHUNK)
    cidx2 = category_seq.astype(jnp.int32).T.reshape(N // CHUNK, CHUNK)
    packed = _sc_gather_sum(poi_table, cat_table, D, pidx2, cidx2, K=8)

    hourT3 = hour_seq.astype(jnp.int32).T.reshape(L_, 1, B_)
    hour_pad = jnp.pad(hour_table, ((0, 32 - hour_table.shape[0]), (0, 0)))
    outT = _tc_dense(packed, hourT3, W, hour_pad, b, B_, L_)
    return outT.reshape(L_, D, B_).transpose(2, 0, 1)
